# baseline (speedup 1.0000x reference)
import math

import jax
import jax.numpy as jnp
from jax.experimental import pallas as pl
from jax.experimental.pallas import tpu as pltpu

CS = 8
NBINS = 9
H = 384
W = 384
NC = H // CS
IMGS = 8

def _hist_one(xb, P, PT, perm):
    gray = 0.2989 * xb[0] + 0.587 * xb[1] + 0.114 * xb[2]
    gray = gray.astype(jnp.bfloat16).astype(jnp.float32)

    zrow = jnp.zeros((1, W), dtype=jnp.float32)
    zcol = jnp.zeros((H, 1), dtype=jnp.float32)
    gl = jnp.concatenate([gray[:, 1:], zcol], axis=1)
    gr = jnp.concatenate([zcol, gray[:, :W - 1]], axis=1)
    d = gl - gr
    s = gr + 2.0 * gray + gl
    d_up = jnp.concatenate([d[1:], zrow], axis=0)
    d_dn = jnp.concatenate([zrow, d[:H - 1]], axis=0)
    s_up = jnp.concatenate([s[1:], zrow], axis=0)
    s_dn = jnp.concatenate([zrow, s[:H - 1]], axis=0)
    gx = d_dn + 2.0 * d + d_up
    gy = s_up - s_dn

    mag = jnp.sqrt((gx * gx + gy * gy + 1e-6).astype(jnp.bfloat16)).astype(jnp.float32)

    zero = jnp.zeros_like(mag)
    mag_u = jnp.where(gy >= 0.0, mag, zero)
    mag_l = mag - mag_u
    cum = []
    for k in range(1, NBINS):
        th = 2.0 * math.pi * k / NBINS
        t = math.cos(th) * gy - math.sin(th) * gx
        src = mag_u if k <= 4 else mag_l
        cum.append(jnp.where(t >= 0.0, src, zero))
    planes = [
        mag_u - cum[0],
        cum[0] - cum[1],
        cum[1] - cum[2],
        cum[2] - cum[3],
        cum[3] + (mag_l - cum[4]),
        cum[4] - cum[5],
        cum[5] - cum[6],
        cum[6] - cum[7],
        cum[7],
    ]
    cols = []
    for b in range(NBINS):
        mb = planes[b].astype(jnp.bfloat16)
        cols.append(jnp.dot(mb, P, preferred_element_type=jnp.float32))
    ccat = jnp.concatenate(cols, axis=1)
    hh = jnp.dot(PT, ccat, preferred_element_type=jnp.float32)
    hp = jnp.dot(hh, perm, preferred_element_type=jnp.float32)

    ss = jnp.sum(hp * hp)
    return hp / (jnp.sqrt(ss) + 1e-6)


def _hog_body(x_ref, o_ref):
    ri = jax.lax.broadcasted_iota(jnp.int32, (H, NC), 0)
    ci = jax.lax.broadcasted_iota(jnp.int32, (H, NC), 1)
    P = (ri // CS == ci).astype(jnp.bfloat16)
    rit = jax.lax.broadcasted_iota(jnp.int32, (NC, H), 0)
    cit = jax.lax.broadcasted_iota(jnp.int32, (NC, H), 1)
    PT = (cit // CS == rit).astype(jnp.float32)

    NW = NC * NBINS
    rp = jax.lax.broadcasted_iota(jnp.int32, (NW, NW), 0)
    cp = jax.lax.broadcasted_iota(jnp.int32, (NW, NW), 1)
    perm = (cp == (rp % NC) * NBINS + rp // NC).astype(jnp.float32)

    for i in range(IMGS):
        o_ref[i] = _hist_one(x_ref[i], P, PT, perm)


def kernel(x):
    B = x.shape[0]
    out = pl.pallas_call(
        _hog_body,
        grid=(B // IMGS,),
        in_specs=[pl.BlockSpec((IMGS, 3, H, W), lambda b: (b, 0, 0, 0))],
        out_specs=pl.BlockSpec((IMGS, NC, NC * NBINS), lambda b: (b, 0, 0)),
        out_shape=jax.ShapeDtypeStruct((B, NC, NC * NBINS), jnp.float32),
        compiler_params=pltpu.CompilerParams(
            dimension_semantics=("parallel",)),
    )(x)
    return out.reshape(B, NC * NC * NBINS)
